# baseline (speedup 1.0000x reference)
import functools

import jax
import jax.numpy as jnp
from jax import lax
from jax.experimental import pallas as pl
from jax.experimental.pallas import tpu as pltpu
from jax.experimental.pallas import tpu_sc as plsc

N_NODES = 10000
D = 128
NC = 2
NS = 16
NW = NC * NS
CHUNK = 128
DEG_W = 128
DEG_PAD = 10240
TRASH = 240


def _deg_kernel_core(dst_hbm, ones_hbm, zeros_hbm, out_hbm,
                     dst_slab, ones_v, zero_v, deg_sh):
    c = lax.axis_index("c")
    s = lax.axis_index("s")
    w = s * NC + c
    nchunks = dst_slab.shape[0]
    rows_per_tile = DEG_PAD // NS
    base = s * rows_per_tile
    bb = zero_v.shape[0]

    pltpu.sync_copy(ones_hbm, ones_v)
    pltpu.sync_copy(zeros_hbm, zero_v)
    for k in range(rows_per_tile // bb):
        pltpu.sync_copy(zero_v, deg_sh.at[pl.ds(base + k * bb, bb)])
    pltpu.sync_copy(dst_hbm.at[w], dst_slab)
    plsc.subcore_barrier()

    def chunk_i(j, _):
        pltpu.sync_copy(ones_v, deg_sh.at[dst_slab.at[j]], add=True)
        return 0

    lax.fori_loop(0, nchunks, chunk_i, 0)
    plsc.subcore_barrier()
    for k in range(rows_per_tile // bb):
        pltpu.sync_copy(deg_sh.at[pl.ds(base + k * bb, bb)], zero_v)
        pltpu.sync_copy(zero_v, out_hbm.at[c, pl.ds(base + k * bb, bb)])


def _agg_kernel_core(h_hbm, x_hbm, src_hbm, dst_hbm, out_hbm,
                     src_slab, dst_slab, rows_a, rows_b,
                     gsem_a, gsem_b, ssem_a, ssem_b, agg_sh):
    c = lax.axis_index("c")
    s = lax.axis_index("s")
    w = s * NC + c
    nchunks = src_hbm.shape[1]
    n = x_hbm.shape[0]
    rows_per_tile = 640
    base = jnp.minimum(s * rows_per_tile, n - rows_per_tile)
    bb = 80
    bounce = rows_a.at[pl.ds(0, bb)]

    for k in range(rows_per_tile // bb):
        pltpu.sync_copy(x_hbm.at[pl.ds(base + k * bb, bb)], bounce)
        pltpu.sync_copy(bounce, agg_sh.at[pl.ds(base + k * bb, bb)])
    plsc.subcore_barrier()

    nh2 = src_slab.shape[0]
    npairs = nh2 // 2

    def wait_gather(slab_row, buf, sem):
        pltpu.make_async_copy(h_hbm.at[slab_row], buf, sem).wait()

    def wait_scatter(buf, slab_row, sem):
        pltpu.make_async_copy(buf, agg_sh.at[slab_row], sem).wait()

    for half in range(nchunks // nh2):
        pltpu.sync_copy(src_hbm.at[w, pl.ds(half * nh2, nh2)], src_slab)
        pltpu.sync_copy(dst_hbm.at[w, pl.ds(half * nh2, nh2)], dst_slab)
        pltpu.async_copy(h_hbm.at[src_slab.at[0]], rows_a, gsem_a)

        @pl.loop(0, npairs - 1, unroll=2)
        def pair_i(j):
            ca = 2 * j
            cb = 2 * j + 1
            pltpu.async_copy(h_hbm.at[src_slab.at[cb]], rows_b, gsem_b)
            wait_gather(src_slab.at[ca], rows_a, gsem_a)
            pltpu.async_copy(rows_a, agg_sh.at[dst_slab.at[ca]], ssem_a,
                             add=True)
            wait_gather(src_slab.at[cb], rows_b, gsem_b)
            wait_scatter(rows_a, dst_slab.at[ca], ssem_a)
            pltpu.async_copy(h_hbm.at[src_slab.at[ca + 2]], rows_a, gsem_a)
            pltpu.async_copy(rows_b, agg_sh.at[dst_slab.at[cb]], ssem_b,
                             add=True)
            wait_scatter(rows_b, dst_slab.at[cb], ssem_b)

        ca = nh2 - 2
        cb = nh2 - 1
        pltpu.async_copy(h_hbm.at[src_slab.at[cb]], rows_b, gsem_b)
        wait_gather(src_slab.at[ca], rows_a, gsem_a)
        pltpu.async_copy(rows_a, agg_sh.at[dst_slab.at[ca]], ssem_a, add=True)
        wait_gather(src_slab.at[cb], rows_b, gsem_b)
        wait_scatter(rows_a, dst_slab.at[ca], ssem_a)
        pltpu.async_copy(rows_b, agg_sh.at[dst_slab.at[cb]], ssem_b, add=True)
        wait_scatter(rows_b, dst_slab.at[cb], ssem_b)
    plsc.subcore_barrier()
    for k in range(rows_per_tile // bb):
        pltpu.sync_copy(agg_sh.at[pl.ds(base + k * bb, bb)], bounce)
        pltpu.sync_copy(bounce, out_hbm.at[c, pl.ds(base + k * bb, bb)])


def _h_body(x_ref, degp_ref, base_ref, h_ref):
    d = degp_ref[0] + degp_ref[1]
    deg = d[:, 0:1]
    dis = lax.rsqrt(deg + base_ref[0, 0] + 1.0)
    h_ref[...] = x_ref[...] * dis


def _out_body(parts_ref, x_ref, degp_ref, w_ref, base_ref, o_ref):
    d = degp_ref[0] + degp_ref[1]
    deg = d[:, 0:1]
    dis = lax.rsqrt(deg + base_ref[0, 0] + 1.0)
    agg = (parts_ref[0] + parts_ref[1] - x_ref[...]) * dis
    y = lax.dot_general(agg, w_ref[...], (((1,), (1,)), ((), ())),
                        preferred_element_type=jnp.float32)
    o_ref[...] = jnp.maximum(y, 0.0)


def kernel(x, edge_index, N, W):
    n = x.shape[0]
    deg_base = (jnp.asarray(N) - n).astype(jnp.float32).reshape(1, 1)
    e = edge_index.shape[1]
    nchunks = -(-e // (NW * CHUNK))
    nchunks += (-nchunks) % 4
    pad = NW * nchunks * CHUNK - e
    src_flat = edge_index[0].astype(jnp.int32)
    dst_flat = edge_index[1].astype(jnp.int32)
    pad_i = jnp.arange(pad, dtype=jnp.int32)
    src = jnp.concatenate([src_flat, pad_i % n]).reshape(NW, nchunks, CHUNK)
    dst = jnp.concatenate([dst_flat, n + pad_i % TRASH]).reshape(
        NW, nchunks, CHUNK)

    mesh = plsc.VectorSubcoreMesh(core_axis_name="c", subcore_axis_name="s")

    deg_call = pl.kernel(
        _deg_kernel_core,
        out_type=jax.ShapeDtypeStruct((NC, DEG_PAD, DEG_W), jnp.float32),
        mesh=mesh,
        scratch_types=[
            pltpu.VMEM((nchunks, CHUNK), jnp.int32),
            pltpu.VMEM((CHUNK, DEG_W), jnp.float32),
            pltpu.VMEM((80, DEG_W), jnp.float32),
            pltpu.VMEM_SHARED((DEG_PAD, DEG_W), jnp.float32),
        ],
    )
    ones_in = jnp.ones((CHUNK, DEG_W), jnp.float32)
    zeros_in = jnp.zeros((80, DEG_W), jnp.float32)
    degp = deg_call(dst, ones_in, zeros_in)

    RB = 1000
    grid = n // RB
    h = pl.pallas_call(
        _h_body,
        grid=(grid,),
        in_specs=[
            pl.BlockSpec((RB, D), lambda i: (i, 0)),
            pl.BlockSpec((NC, RB, DEG_W), lambda i: (0, i, 0)),
            pl.BlockSpec((1, 1), lambda i: (0, 0)),
        ],
        out_specs=pl.BlockSpec((RB, D), lambda i: (i, 0)),
        out_shape=jax.ShapeDtypeStruct((n, D), jnp.float32),
    )(x, degp, deg_base)

    agg_call = pl.kernel(
        _agg_kernel_core,
        out_type=jax.ShapeDtypeStruct((NC, n, D), jnp.float32),
        mesh=mesh,
        scratch_types=[
            pltpu.VMEM((nchunks // 2, CHUNK), jnp.int32),
            pltpu.VMEM((nchunks // 2, CHUNK), jnp.int32),
            pltpu.VMEM((CHUNK, D), jnp.float32),
            pltpu.VMEM((CHUNK, D), jnp.float32),
            pltpu.SemaphoreType.DMA,
            pltpu.SemaphoreType.DMA,
            pltpu.SemaphoreType.DMA,
            pltpu.SemaphoreType.DMA,
            pltpu.VMEM_SHARED((n + TRASH, D), jnp.float32),
        ],
    )
    parts = agg_call(h, x, src, dst)

    out = pl.pallas_call(
        _out_body,
        grid=(grid,),
        in_specs=[
            pl.BlockSpec((NC, RB, D), lambda i: (0, i, 0)),
            pl.BlockSpec((RB, D), lambda i: (i, 0)),
            pl.BlockSpec((NC, RB, DEG_W), lambda i: (0, i, 0)),
            pl.BlockSpec((D, D), lambda i: (0, 0)),
            pl.BlockSpec((1, 1), lambda i: (0, 0)),
        ],
        out_specs=pl.BlockSpec((RB, D), lambda i: (i, 0)),
        out_shape=jax.ShapeDtypeStruct((n, D), jnp.float32),
    )(parts, x, degp, W, deg_base)
    return out


# speedup vs baseline: 1.2318x; 1.2318x over previous
import functools

import jax
import jax.numpy as jnp
from jax import lax
from jax.experimental import pallas as pl
from jax.experimental.pallas import tpu as pltpu
from jax.experimental.pallas import tpu_sc as plsc

N_NODES = 10000
D = 128
NC = 2
NS = 16
NW = NC * NS
CHUNK = 128
DEG_W = 16
DEG_PAD = 10240
TRASH = 240


def _deg_kernel_core(dst_hbm, ones_hbm, zeros_hbm, out_hbm,
                     dst_slab, ones_v, zero_v, deg_sh):
    c = lax.axis_index("c")
    s = lax.axis_index("s")
    w = s * NC + c
    nchunks = dst_slab.shape[0]
    rows_per_tile = DEG_PAD // NS
    base = s * rows_per_tile
    bb = zero_v.shape[0]

    pltpu.sync_copy(ones_hbm, ones_v)
    pltpu.sync_copy(zeros_hbm, zero_v)
    for k in range(rows_per_tile // bb):
        pltpu.sync_copy(zero_v, deg_sh.at[pl.ds(base + k * bb, bb)])
    pltpu.sync_copy(dst_hbm.at[w], dst_slab)
    plsc.subcore_barrier()

    def chunk_i(j, _):
        pltpu.sync_copy(ones_v, deg_sh.at[dst_slab.at[j]], add=True)
        return 0

    lax.fori_loop(0, nchunks, chunk_i, 0)
    plsc.subcore_barrier()
    for k in range(rows_per_tile // bb):
        pltpu.sync_copy(deg_sh.at[pl.ds(base + k * bb, bb)], zero_v)
        pltpu.sync_copy(zero_v, out_hbm.at[c, pl.ds(base + k * bb, bb)])


def _agg_kernel_core(h_hbm, x_hbm, src_hbm, dst_hbm, out_hbm,
                     src_slab, dst_slab, rows_a, rows_b,
                     gsem_a, gsem_b, ssem_a, ssem_b, agg_sh):
    c = lax.axis_index("c")
    s = lax.axis_index("s")
    w = s * NC + c
    nchunks = src_hbm.shape[1]
    n = x_hbm.shape[0]
    rows_per_tile = 640
    base = jnp.minimum(s * rows_per_tile, n - rows_per_tile)
    bb = 80
    bounce = rows_a.at[pl.ds(0, bb)]

    for k in range(rows_per_tile // bb):
        pltpu.sync_copy(x_hbm.at[pl.ds(base + k * bb, bb)], bounce)
        pltpu.sync_copy(bounce, agg_sh.at[pl.ds(base + k * bb, bb)])
    plsc.subcore_barrier()

    nh2 = src_slab.shape[0]
    npairs = nh2 // 2

    def wait_gather(slab_row, buf, sem):
        pltpu.make_async_copy(h_hbm.at[slab_row], buf, sem).wait()

    def wait_scatter(buf, slab_row, sem):
        pltpu.make_async_copy(buf, agg_sh.at[slab_row], sem).wait()

    for half in range(nchunks // nh2):
        pltpu.sync_copy(src_hbm.at[w, pl.ds(half * nh2, nh2)], src_slab)
        pltpu.sync_copy(dst_hbm.at[w, pl.ds(half * nh2, nh2)], dst_slab)
        pltpu.async_copy(h_hbm.at[src_slab.at[0]], rows_a, gsem_a)

        @pl.loop(0, npairs - 1, unroll=2)
        def pair_i(j):
            ca = 2 * j
            cb = 2 * j + 1
            pltpu.async_copy(h_hbm.at[src_slab.at[cb]], rows_b, gsem_b)
            wait_gather(src_slab.at[ca], rows_a, gsem_a)
            pltpu.async_copy(rows_a, agg_sh.at[dst_slab.at[ca]], ssem_a,
                             add=True)
            wait_gather(src_slab.at[cb], rows_b, gsem_b)
            wait_scatter(rows_a, dst_slab.at[ca], ssem_a)
            pltpu.async_copy(h_hbm.at[src_slab.at[ca + 2]], rows_a, gsem_a)
            pltpu.async_copy(rows_b, agg_sh.at[dst_slab.at[cb]], ssem_b,
                             add=True)
            wait_scatter(rows_b, dst_slab.at[cb], ssem_b)

        ca = nh2 - 2
        cb = nh2 - 1
        pltpu.async_copy(h_hbm.at[src_slab.at[cb]], rows_b, gsem_b)
        wait_gather(src_slab.at[ca], rows_a, gsem_a)
        pltpu.async_copy(rows_a, agg_sh.at[dst_slab.at[ca]], ssem_a, add=True)
        wait_gather(src_slab.at[cb], rows_b, gsem_b)
        wait_scatter(rows_a, dst_slab.at[ca], ssem_a)
        pltpu.async_copy(rows_b, agg_sh.at[dst_slab.at[cb]], ssem_b, add=True)
        wait_scatter(rows_b, dst_slab.at[cb], ssem_b)
    plsc.subcore_barrier()
    for k in range(rows_per_tile // bb):
        pltpu.sync_copy(agg_sh.at[pl.ds(base + k * bb, bb)], bounce)
        pltpu.sync_copy(bounce, out_hbm.at[c, pl.ds(base + k * bb, bb)])


def _h_body(x_ref, degp_ref, base_ref, h_ref):
    d = degp_ref[0, :, 0:1] + degp_ref[1, :, 0:1]
    dis = lax.rsqrt(d + base_ref[0, 0] + 1.0)
    h_ref[...] = x_ref[...] * dis


def _out_body(parts_ref, x_ref, degp_ref, w_ref, base_ref, o_ref):
    d = degp_ref[0, :, 0:1] + degp_ref[1, :, 0:1]
    dis = lax.rsqrt(d + base_ref[0, 0] + 1.0)
    agg = (parts_ref[0] + parts_ref[1] - x_ref[...]) * dis
    y = lax.dot_general(agg, w_ref[...], (((1,), (1,)), ((), ())),
                        preferred_element_type=jnp.float32)
    o_ref[...] = jnp.maximum(y, 0.0)


def kernel(x, edge_index, N, W):
    n = x.shape[0]
    deg_base = (jnp.asarray(N) - n).astype(jnp.float32).reshape(1, 1)
    e = edge_index.shape[1]
    nchunks = -(-e // (NW * CHUNK))
    nchunks += (-nchunks) % 4
    pad = NW * nchunks * CHUNK - e
    src_flat = edge_index[0].astype(jnp.int32)
    dst_flat = edge_index[1].astype(jnp.int32)
    pad_i = jnp.arange(pad, dtype=jnp.int32)
    src = jnp.concatenate([src_flat, pad_i % n]).reshape(NW, nchunks, CHUNK)
    dst = jnp.concatenate([dst_flat, n + pad_i % TRASH]).reshape(
        NW, nchunks, CHUNK)

    mesh = plsc.VectorSubcoreMesh(core_axis_name="c", subcore_axis_name="s")

    deg_call = pl.kernel(
        _deg_kernel_core,
        out_type=jax.ShapeDtypeStruct((NC, DEG_PAD, DEG_W), jnp.float32),
        mesh=mesh,
        compiler_params=pltpu.CompilerParams(use_tc_tiling_on_sc=False),
        scratch_types=[
            pltpu.VMEM((nchunks, CHUNK), jnp.int32),
            pltpu.VMEM((CHUNK, DEG_W), jnp.float32),
            pltpu.VMEM((80, DEG_W), jnp.float32),
            pltpu.VMEM_SHARED((DEG_PAD, DEG_W), jnp.float32),
        ],
    )
    ones_in = jnp.ones((CHUNK, DEG_W), jnp.float32)
    zeros_in = jnp.zeros((80, DEG_W), jnp.float32)
    degp = deg_call(dst, ones_in, zeros_in)

    RB = 1000
    grid = n // RB
    h = pl.pallas_call(
        _h_body,
        grid=(grid,),
        in_specs=[
            pl.BlockSpec((RB, D), lambda i: (i, 0)),
            pl.BlockSpec((NC, RB, DEG_W), lambda i: (0, i, 0)),
            pl.BlockSpec((1, 1), lambda i: (0, 0)),
        ],
        out_specs=pl.BlockSpec((RB, D), lambda i: (i, 0)),
        out_shape=jax.ShapeDtypeStruct((n, D), jnp.float32),
    )(x, degp, deg_base)

    agg_call = pl.kernel(
        _agg_kernel_core,
        out_type=jax.ShapeDtypeStruct((NC, n, D), jnp.float32),
        mesh=mesh,
        scratch_types=[
            pltpu.VMEM((nchunks // 2, CHUNK), jnp.int32),
            pltpu.VMEM((nchunks // 2, CHUNK), jnp.int32),
            pltpu.VMEM((CHUNK, D), jnp.float32),
            pltpu.VMEM((CHUNK, D), jnp.float32),
            pltpu.SemaphoreType.DMA,
            pltpu.SemaphoreType.DMA,
            pltpu.SemaphoreType.DMA,
            pltpu.SemaphoreType.DMA,
            pltpu.VMEM_SHARED((n + TRASH, D), jnp.float32),
        ],
    )
    parts = agg_call(h, x, src, dst)

    out = pl.pallas_call(
        _out_body,
        grid=(grid,),
        in_specs=[
            pl.BlockSpec((NC, RB, D), lambda i: (0, i, 0)),
            pl.BlockSpec((RB, D), lambda i: (i, 0)),
            pl.BlockSpec((NC, RB, DEG_W), lambda i: (0, i, 0)),
            pl.BlockSpec((D, D), lambda i: (0, 0)),
            pl.BlockSpec((1, 1), lambda i: (0, 0)),
        ],
        out_specs=pl.BlockSpec((RB, D), lambda i: (i, 0)),
        out_shape=jax.ShapeDtypeStruct((n, D), jnp.float32),
    )(parts, x, degp, W, deg_base)
    return out
